# Initial kernel scaffold; baseline (speedup 1.0000x reference)
import jax
import jax.numpy as jnp
from jax.experimental import pallas as pl


def kernel(x, edge_index, edge_attr, W1, att_src1, att_dst1, We1, att_e1, b1, W2, att_src2, att_dst2, We2, att_e2, b2, Wl, bl):
    raise NotImplementedError("write your pallas kernel here")


# speedup vs baseline: 1.7680x; 1.7680x over previous
import jax
import jax.numpy as jnp
from jax.experimental import pallas as pl
from jax.experimental.pallas import tpu as pltpu

N_NODES = 10000
N_EDGES = 320000
D_FEAT = 128
HID = 32


def _matmul_kernel(x_ref, w_ref, o_ref):
    o_ref[...] = jnp.dot(x_ref[...], w_ref[...], preferred_element_type=jnp.float32)


def _tc_matmul(x, w):
    n, k = x.shape
    m = w.shape[1]
    blk = 1000
    return pl.pallas_call(
        _matmul_kernel,
        grid=(n // blk,),
        in_specs=[
            pl.BlockSpec((blk, k), lambda i: (i, 0)),
            pl.BlockSpec((k, m), lambda i: (0, 0)),
        ],
        out_specs=pl.BlockSpec((blk, m), lambda i: (i, 0)),
        out_shape=jax.ShapeDtypeStruct((n, m), jnp.float32),
    )(x, w)


def _gat_layer(h, src, dst, ae, deg, sae, b):
    N = h.shape[0]
    a_s = h @ ae["att_src"]
    a_d = h @ ae["att_dst"]
    a_e = ae["ae"]
    loop_ae = sae / jnp.clip(deg, 1.0)
    M = jax.nn.leaky_relu(jnp.max(a_s) + jnp.max(a_d) + jnp.maximum(jnp.max(a_e), 0.0), 0.2)
    alpha = jax.nn.leaky_relu(a_s[src] + a_d[dst] + a_e, 0.2)
    alpha_loop = jax.nn.leaky_relu(a_s + a_d + loop_ae, 0.2)
    w = jnp.exp(alpha - M)
    wl = jnp.exp(alpha_loop - M)
    denom = jax.ops.segment_sum(w, dst, num_segments=N) + wl
    num = jax.ops.segment_sum(h[src] * w[:, None], dst, num_segments=N) + wl[:, None] * h
    return num / (denom[:, None] + 1e-16) + b


def kernel(x, edge_index, edge_attr, W1, att_src1, att_dst1, We1, att_e1, b1,
           W2, att_src2, att_dst2, We2, att_e2, b2, Wl, bl):
    src, dst = edge_index[0], edge_index[1]
    N = x.shape[0]
    deg = jax.ops.segment_sum(jnp.ones_like(src, dtype=jnp.float32), dst, num_segments=N)
    wv = jnp.stack([We1 @ att_e1, We2 @ att_e2], axis=1)
    ae12 = edge_attr @ wv
    sae = jax.ops.segment_sum(ae12, dst, num_segments=N)

    h1 = _tc_matmul(x, W1)
    o1 = _gat_layer(h1, src, dst,
                    {"att_src": att_src1, "att_dst": att_dst1, "ae": ae12[:, 0]},
                    deg, sae[:, 0], b1)
    h2 = _tc_matmul(jax.nn.relu(o1), W2)
    o2 = _gat_layer(h2, src, dst,
                    {"att_src": att_src2, "att_dst": att_dst2, "ae": ae12[:, 1]},
                    deg, sae[:, 1], b2)
    out = o2 @ Wl + bl
    return jax.nn.relu(out)


# speedup vs baseline: 19.8346x; 11.2185x over previous
import dataclasses
import functools

import jax
import jax.numpy as jnp
from jax import lax
from jax.experimental import pallas as pl
from jax.experimental.pallas import tpu as pltpu
from jax.experimental.pallas import tpu_sc as plsc

N_NODES = 10000
N_EDGES = 320000
D_FEAT = 128
HID = 32

NC = 2
NS = 16
NW = NC * NS
CH = 128
NCH = 80
EC = NCH * CH
E_PAD = NW * EC
NP = 10240
NPS = NP // NS


def _matmul_kernel(x_ref, w_ref, o_ref):
    o_ref[...] = jnp.dot(x_ref[...], w_ref[...], preferred_element_type=jnp.float32)


def _tc_matmul(x, w):
    n, k = x.shape
    m = w.shape[1]
    blk = 1000
    return pl.pallas_call(
        _matmul_kernel,
        grid=(n // blk,),
        in_specs=[
            pl.BlockSpec((blk, k), lambda i: (i, 0)),
            pl.BlockSpec((k, m), lambda i: (0, 0)),
        ],
        out_specs=pl.BlockSpec((blk, m), lambda i: (i, 0)),
        out_shape=jax.ShapeDtypeStruct((n, m), jnp.float32),
    )(x, w)


_MESH = plsc.VectorSubcoreMesh(core_axis_name="c", subcore_axis_name="s")

_SC_PARAMS = pltpu.CompilerParams()
if "needs_layout_passes" in pltpu.CompilerParams.__dataclass_fields__:
    _SC_PARAMS = dataclasses.replace(_SC_PARAMS, needs_layout_passes=False)
if "use_tc_tiling_on_sc" in pltpu.CompilerParams.__dataclass_fields__:
    _SC_PARAMS = dataclasses.replace(_SC_PARAMS, use_tc_tiling_on_sc=False)


@functools.partial(
    pl.kernel,
    out_type=(jax.ShapeDtypeStruct((NC, NP, HID), jnp.float32),
              jax.ShapeDtypeStruct((NC, NP), jnp.float32)),
    mesh=_MESH,
    compiler_params=_SC_PARAMS,
    scratch_types=[
        pltpu.VMEM((NCH, CH), jnp.int32),
        pltpu.VMEM((NCH, CH), jnp.int32),
        pltpu.VMEM((NCH, CH), jnp.float32),
        pltpu.VMEM((NP,), jnp.float32),
        pltpu.VMEM((NP,), jnp.float32),
        pltpu.VMEM((16,), jnp.float32),
        pltpu.VMEM((CH, HID), jnp.float32),
        pltpu.VMEM_SHARED((NP, HID), jnp.float32),
        pltpu.VMEM_SHARED((NP,), jnp.float32),
        pltpu.SemaphoreType.DMA,
    ],
)
def _sc_layer_kernel(src_hbm, dst_hbm, ae_hbm, as_hbm, ad_hbm, m_hbm, h_hbm,
                     z2_hbm, z1_hbm, num_out, den_out,
                     src_v, dst_v, w_v, as_v, ad_v, m_v, rows_v,
                     num_sh, den_sh, sem):
    cid = lax.axis_index("c")
    sid = lax.axis_index("s")
    wid = cid * NS + sid

    pltpu.sync_copy(z2_hbm, num_sh.at[pl.ds(sid * NPS, NPS)])
    pltpu.sync_copy(z1_hbm, den_sh.at[pl.ds(sid * NPS, NPS)])

    pltpu.sync_copy(src_hbm.at[wid], src_v)
    pltpu.sync_copy(dst_hbm.at[wid], dst_v)
    pltpu.sync_copy(ae_hbm.at[wid], w_v)
    pltpu.sync_copy(as_hbm, as_v)
    pltpu.sync_copy(ad_hbm, ad_v)
    pltpu.sync_copy(m_hbm, m_v)
    plsc.subcore_barrier()

    mvec = m_v[...]

    @pl.loop(0, NCH)
    def _alpha(g):
        @pl.loop(0, CH // 16)
        def _alpha_inner(k):
            o = k * 16
            s16 = src_v[g, pl.ds(o, 16)]
            d16 = dst_v[g, pl.ds(o, 16)]
            ae16 = w_v[g, pl.ds(o, 16)]
            t = plsc.load_gather(as_v, [s16]) + plsc.load_gather(ad_v, [d16]) + ae16
            al = jnp.maximum(t, 0.2 * t)
            w_v[g, pl.ds(o, 16)] = jnp.exp(al - mvec)

    @pl.loop(0, NCH)
    def _agg(c):
        pltpu.async_copy(h_hbm.at[src_v.at[c]], rows_v, sem).wait()

        @pl.loop(0, CH)
        def _scale(e):
            wv = plsc.load_gather(
                w_v, [jnp.full((16,), c, jnp.int32), jnp.full((16,), e, jnp.int32)])
            rows_v[e, pl.ds(0, 16)] = rows_v[e, pl.ds(0, 16)] * wv
            rows_v[e, pl.ds(16, 16)] = rows_v[e, pl.ds(16, 16)] * wv

        pltpu.sync_copy(rows_v, num_sh.at[dst_v.at[c]], add=True)
        pltpu.sync_copy(w_v.at[c], den_sh.at[dst_v.at[c]], add=True)

    plsc.subcore_barrier()

    pltpu.sync_copy(num_sh.at[pl.ds(sid * NPS, NPS)],
                    num_out.at[cid, pl.ds(sid * NPS, NPS)])
    pltpu.sync_copy(den_sh.at[pl.ds(sid * NPS, NPS)],
                    den_out.at[cid, pl.ds(sid * NPS, NPS)])


@functools.partial(
    pl.kernel,
    out_type=(jax.ShapeDtypeStruct((NC, NP), jnp.float32),
              jax.ShapeDtypeStruct((NC, NP), jnp.float32),
              jax.ShapeDtypeStruct((NC, NP), jnp.float32)),
    mesh=_MESH,
    compiler_params=_SC_PARAMS,
    scratch_types=[
        pltpu.VMEM((NCH, CH), jnp.int32),
        pltpu.VMEM((NCH, CH), jnp.float32),
        pltpu.VMEM((NCH, CH), jnp.float32),
        pltpu.VMEM((NCH, CH), jnp.float32),
        pltpu.VMEM_SHARED((NP,), jnp.float32),
        pltpu.VMEM_SHARED((NP,), jnp.float32),
        pltpu.VMEM_SHARED((NP,), jnp.float32),
    ],
)
def _sc_deg_kernel(dst_hbm, ones_hbm, ae1_hbm, ae2_hbm, z1_hbm,
                   deg_out, s1_out, s2_out,
                   dst_v, on_v, a1_v, a2_v, deg_sh, s1_sh, s2_sh):
    cid = lax.axis_index("c")
    sid = lax.axis_index("s")
    wid = cid * NS + sid

    pltpu.sync_copy(z1_hbm, deg_sh.at[pl.ds(sid * NPS, NPS)])
    pltpu.sync_copy(z1_hbm, s1_sh.at[pl.ds(sid * NPS, NPS)])
    pltpu.sync_copy(z1_hbm, s2_sh.at[pl.ds(sid * NPS, NPS)])

    pltpu.sync_copy(dst_hbm.at[wid], dst_v)
    pltpu.sync_copy(ones_hbm.at[wid], on_v)
    pltpu.sync_copy(ae1_hbm.at[wid], a1_v)
    pltpu.sync_copy(ae2_hbm.at[wid], a2_v)
    plsc.subcore_barrier()

    @pl.loop(0, NCH)
    def _scat(c):
        pltpu.sync_copy(on_v.at[c], deg_sh.at[dst_v.at[c]], add=True)
        pltpu.sync_copy(a1_v.at[c], s1_sh.at[dst_v.at[c]], add=True)
        pltpu.sync_copy(a2_v.at[c], s2_sh.at[dst_v.at[c]], add=True)

    plsc.subcore_barrier()

    pltpu.sync_copy(deg_sh.at[pl.ds(sid * NPS, NPS)],
                    deg_out.at[cid, pl.ds(sid * NPS, NPS)])
    pltpu.sync_copy(s1_sh.at[pl.ds(sid * NPS, NPS)],
                    s1_out.at[cid, pl.ds(sid * NPS, NPS)])
    pltpu.sync_copy(s2_sh.at[pl.ds(sid * NPS, NPS)],
                    s2_out.at[cid, pl.ds(sid * NPS, NPS)])


def _pad_edges(a, fill):
    pad = E_PAD - N_EDGES
    return jnp.concatenate([a, jnp.full((pad,), fill, a.dtype)]).reshape(NW, NCH, CH)


def _sc_layer(src3, dst3, ae_att3, a_s, a_d, h, z2, z1):
    N = N_NODES
    as_p = jnp.concatenate([a_s, jnp.zeros((NP - N,), jnp.float32)])
    ad_p = jnp.concatenate([a_d, jnp.zeros((NP - N,), jnp.float32)])
    m = jax.nn.leaky_relu(jnp.max(a_s) + jnp.max(a_d)
                          + jnp.maximum(jnp.max(ae_att3), 0.0), 0.2)
    m16 = jnp.full((16,), m, jnp.float32)
    num_p, den_p = _sc_layer_kernel(src3, dst3, ae_att3, as_p, ad_p, m16, h, z2, z1)
    num = (num_p[0] + num_p[1])[:N]
    den = (den_p[0] + den_p[1])[:N]
    return num, den, m


def kernel(x, edge_index, edge_attr, W1, att_src1, att_dst1, We1, att_e1, b1,
           W2, att_src2, att_dst2, We2, att_e2, b2, Wl, bl):
    N = N_NODES
    src, dst = edge_index[0], edge_index[1]
    src3 = _pad_edges(src, 0)
    dst3 = _pad_edges(dst, 0)

    wv = jnp.stack([We1 @ att_e1, We2 @ att_e2], axis=1)
    ae12 = _tc_matmul(edge_attr, wv)
    ae1_att3 = _pad_edges(ae12[:, 0], -1e30)
    ae2_att3 = _pad_edges(ae12[:, 1], -1e30)
    ae1_sum3 = _pad_edges(ae12[:, 0], 0.0)
    ae2_sum3 = _pad_edges(ae12[:, 1], 0.0)
    ones3 = _pad_edges(jnp.ones((N_EDGES,), jnp.float32), 0.0)

    z2 = jnp.zeros((NPS, HID), jnp.float32)
    z1 = jnp.zeros((NPS,), jnp.float32)

    deg_p, s1_p, s2_p = _sc_deg_kernel(dst3, ones3, ae1_sum3, ae2_sum3, z1)
    deg = (deg_p[0] + deg_p[1])[:N]
    lae1 = ((s1_p[0] + s1_p[1])[:N]) / jnp.clip(deg, 1.0)
    lae2 = ((s2_p[0] + s2_p[1])[:N]) / jnp.clip(deg, 1.0)

    h1 = _tc_matmul(x, W1)
    as1 = h1 @ att_src1
    ad1 = h1 @ att_dst1
    num1, den1, m1 = _sc_layer(src3, dst3, ae1_att3, as1, ad1, h1, z2, z1)
    wl1 = jnp.exp(jax.nn.leaky_relu(as1 + ad1 + lae1, 0.2) - m1)
    o1 = (num1 + wl1[:, None] * h1) / (den1[:, None] + wl1[:, None] + 1e-16) + b1
    o1 = jax.nn.relu(o1)

    h2 = _tc_matmul(o1, W2)
    as2 = h2 @ att_src2
    ad2 = h2 @ att_dst2
    num2, den2, m2 = _sc_layer(src3, dst3, ae2_att3, as2, ad2, h2, z2, z1)
    wl2 = jnp.exp(jax.nn.leaky_relu(as2 + ad2 + lae2, 0.2) - m2)
    o2 = (num2 + wl2[:, None] * h2) / (den2[:, None] + wl2[:, None] + 1e-16) + b2

    out = o2 @ Wl + bl
    return jax.nn.relu(out)


# speedup vs baseline: 21.4775x; 1.0828x over previous
import dataclasses
import functools

import jax
import jax.numpy as jnp
from jax import lax
from jax.experimental import pallas as pl
from jax.experimental.pallas import tpu as pltpu
from jax.experimental.pallas import tpu_sc as plsc

N_NODES = 10000
N_EDGES = 320000
D_FEAT = 128
HID = 32

NC = 2
NS = 16
NW = NC * NS
CH = 128
NCH = 80
EC = NCH * CH
E_PAD = NW * EC
NP = 10240
NPS = NP // NS


def _matmul_kernel(x_ref, w_ref, o_ref):
    o_ref[...] = jnp.dot(x_ref[...], w_ref[...], preferred_element_type=jnp.float32,
                         precision=jax.lax.Precision.HIGHEST)


def _tc_matmul(x, w):
    n, k = x.shape
    m = w.shape[1]
    blk = 1000
    return pl.pallas_call(
        _matmul_kernel,
        grid=(n // blk,),
        in_specs=[
            pl.BlockSpec((blk, k), lambda i: (i, 0)),
            pl.BlockSpec((k, m), lambda i: (0, 0)),
        ],
        out_specs=pl.BlockSpec((blk, m), lambda i: (i, 0)),
        out_shape=jax.ShapeDtypeStruct((n, m), jnp.float32),
    )(x, w)


_MESH = plsc.VectorSubcoreMesh(core_axis_name="c", subcore_axis_name="s")

_SC_PARAMS = pltpu.CompilerParams()
if "needs_layout_passes" in pltpu.CompilerParams.__dataclass_fields__:
    _SC_PARAMS = dataclasses.replace(_SC_PARAMS, needs_layout_passes=False)
if "use_tc_tiling_on_sc" in pltpu.CompilerParams.__dataclass_fields__:
    _SC_PARAMS = dataclasses.replace(_SC_PARAMS, use_tc_tiling_on_sc=False)


def _make_sc_layer(with_deg):
    n_extra = 3 if with_deg else 0
    out_type = tuple([jax.ShapeDtypeStruct((NC, NP, HID), jnp.float32)]
                     + [jax.ShapeDtypeStruct((NC, NP), jnp.float32)] * (1 + n_extra))
    scratch = [
        pltpu.VMEM((NCH, CH), jnp.int32),
        pltpu.VMEM((NCH, CH), jnp.int32),
        pltpu.VMEM((NCH, CH), jnp.float32),
        pltpu.VMEM((N_NODES,), jnp.float32),
        pltpu.VMEM((N_NODES,), jnp.float32),
        pltpu.VMEM((16,), jnp.float32),
        pltpu.VMEM((CH, HID), jnp.float32),
        pltpu.VMEM((CH, HID), jnp.float32),
        pltpu.VMEM_SHARED((NP, HID), jnp.float32),
        pltpu.VMEM_SHARED((NP,), jnp.float32),
        pltpu.SemaphoreType.DMA,
        pltpu.SemaphoreType.DMA,
    ]
    if with_deg:
        scratch += [
            pltpu.VMEM((NCH, CH), jnp.float32),
            pltpu.VMEM((NCH, CH), jnp.float32),
            pltpu.VMEM((NCH, CH), jnp.float32),
            pltpu.VMEM_SHARED((NP,), jnp.float32),
            pltpu.VMEM_SHARED((NP,), jnp.float32),
            pltpu.VMEM_SHARED((NP,), jnp.float32),
        ]

    def body(*refs):
        if with_deg:
            (src_hbm, dst_hbm, ae_hbm, on_hbm, a1_hbm, a2_hbm,
             as_hbm, ad_hbm, m_hbm, h_hbm, z2_hbm, z1_hbm,
             num_out, den_out, deg_out, s1_out, s2_out,
             src_v, dst_v, w_v, as_v, ad_v, m_v, rows_a, rows_b,
             num_sh, den_sh, gsa, gsb,
             on_v, a1_v, a2_v, deg_sh, s1_sh, s2_sh) = refs
        else:
            (src_hbm, dst_hbm, ae_hbm,
             as_hbm, ad_hbm, m_hbm, h_hbm, z2_hbm, z1_hbm,
             num_out, den_out,
             src_v, dst_v, w_v, as_v, ad_v, m_v, rows_a, rows_b,
             num_sh, den_sh, gsa, gsb) = refs

        cid = lax.axis_index("c")
        sid = lax.axis_index("s")
        wid = cid * NS + sid

        pltpu.sync_copy(z2_hbm, num_sh.at[pl.ds(sid * NPS, NPS)])
        pltpu.sync_copy(z1_hbm, den_sh.at[pl.ds(sid * NPS, NPS)])
        if with_deg:
            pltpu.sync_copy(z1_hbm, deg_sh.at[pl.ds(sid * NPS, NPS)])
            pltpu.sync_copy(z1_hbm, s1_sh.at[pl.ds(sid * NPS, NPS)])
            pltpu.sync_copy(z1_hbm, s2_sh.at[pl.ds(sid * NPS, NPS)])

        pltpu.sync_copy(src_hbm.at[wid], src_v)
        pltpu.sync_copy(dst_hbm.at[wid], dst_v)
        pltpu.sync_copy(ae_hbm.at[wid], w_v)
        pltpu.sync_copy(as_hbm, as_v)
        pltpu.sync_copy(ad_hbm, ad_v)
        pltpu.sync_copy(m_hbm, m_v)
        if with_deg:
            pltpu.sync_copy(on_hbm.at[wid], on_v)
            pltpu.sync_copy(a1_hbm.at[wid], a1_v)
            pltpu.sync_copy(a2_hbm.at[wid], a2_v)
        plsc.subcore_barrier()

        mvec = m_v[...]

        @pl.loop(0, NCH)
        def _alpha(g):
            @pl.loop(0, CH // 16)
            def _alpha_inner(k):
                o = k * 16
                s16 = src_v[g, pl.ds(o, 16)]
                d16 = dst_v[g, pl.ds(o, 16)]
                ae16 = w_v[g, pl.ds(o, 16)]
                t = plsc.load_gather(as_v, [s16]) + plsc.load_gather(ad_v, [d16]) + ae16
                al = jnp.maximum(t, 0.2 * t)
                w_v[g, pl.ds(o, 16)] = jnp.exp(al - mvec)

        @pl.loop(0, NCH)
        def _scal_scat(c):
            pltpu.sync_copy(w_v.at[c], den_sh.at[dst_v.at[c]], add=True)
            if with_deg:
                pltpu.sync_copy(on_v.at[c], deg_sh.at[dst_v.at[c]], add=True)
                pltpu.sync_copy(a1_v.at[c], s1_sh.at[dst_v.at[c]], add=True)
                pltpu.sync_copy(a2_v.at[c], s2_sh.at[dst_v.at[c]], add=True)

        def scale(rows_v, c):
            @pl.loop(0, CH)
            def _scale(e):
                wv = plsc.load_gather(
                    w_v, [jnp.full((16,), c, jnp.int32), jnp.full((16,), e, jnp.int32)])
                rows_v[e, pl.ds(0, 16)] = rows_v[e, pl.ds(0, 16)] * wv
                rows_v[e, pl.ds(16, 16)] = rows_v[e, pl.ds(16, 16)] * wv

        pltpu.async_copy(h_hbm.at[src_v.at[0]], rows_a, gsa)
        pltpu.async_copy(h_hbm.at[src_v.at[1]], rows_b, gsb)

        @pl.loop(0, NCH // 2)
        def _agg(p):
            c0 = p * 2
            c1 = c0 + 1

            pltpu.make_async_copy(h_hbm.at[src_v.at[c0]], rows_a, gsa).wait()
            scale(rows_a, c0)
            pltpu.sync_copy(rows_a, num_sh.at[dst_v.at[c0]], add=True)

            @pl.when(c0 + 2 < NCH)
            def _next_a():
                pltpu.async_copy(h_hbm.at[src_v.at[c0 + 2]], rows_a, gsa)

            pltpu.make_async_copy(h_hbm.at[src_v.at[c1]], rows_b, gsb).wait()
            scale(rows_b, c1)
            pltpu.sync_copy(rows_b, num_sh.at[dst_v.at[c1]], add=True)

            @pl.when(c1 + 2 < NCH)
            def _next_b():
                pltpu.async_copy(h_hbm.at[src_v.at[c1 + 2]], rows_b, gsb)

        plsc.subcore_barrier()

        sl = pl.ds(sid * NPS, NPS)
        pltpu.sync_copy(num_sh.at[sl], num_out.at[cid, sl])
        pltpu.sync_copy(den_sh.at[sl], den_out.at[cid, sl])
        if with_deg:
            pltpu.sync_copy(deg_sh.at[sl], deg_out.at[cid, sl])
            pltpu.sync_copy(s1_sh.at[sl], s1_out.at[cid, sl])
            pltpu.sync_copy(s2_sh.at[sl], s2_out.at[cid, sl])

    return functools.partial(
        pl.kernel, mesh=_MESH, out_type=out_type,
        compiler_params=_SC_PARAMS, scratch_types=scratch)(body)


_SC_LAYER1 = _make_sc_layer(True)
_SC_LAYER2 = _make_sc_layer(False)


def _pad_edges(a, fill):
    pad = E_PAD - N_EDGES
    return jnp.concatenate([a, jnp.full((pad,), fill, a.dtype)]).reshape(NW, NCH, CH)


def _att_scalars(h, att_src, att_dst, ae):
    a_s = h @ att_src
    a_d = h @ att_dst
    m = jax.nn.leaky_relu(jnp.max(a_s) + jnp.max(a_d)
                          + jnp.maximum(jnp.max(ae), 0.0), 0.2)
    return a_s, a_d, m, jnp.full((16,), m, jnp.float32)


def kernel(x, edge_index, edge_attr, W1, att_src1, att_dst1, We1, att_e1, b1,
           W2, att_src2, att_dst2, We2, att_e2, b2, Wl, bl):
    N = N_NODES
    src, dst = edge_index[0], edge_index[1]
    src3 = _pad_edges(src, 0)
    dst3 = _pad_edges(dst, 0)

    wv = jnp.stack([We1 @ att_e1, We2 @ att_e2], axis=1)
    ae12 = _tc_matmul(edge_attr, wv)
    ae1_att3 = _pad_edges(ae12[:, 0], -1e30)
    ae2_att3 = _pad_edges(ae12[:, 1], -1e30)
    ae1_sum3 = _pad_edges(ae12[:, 0], 0.0)
    ae2_sum3 = _pad_edges(ae12[:, 1], 0.0)
    ones3 = _pad_edges(jnp.ones((N_EDGES,), jnp.float32), 0.0)

    z2 = jnp.zeros((NPS, HID), jnp.float32)
    z1 = jnp.zeros((NPS,), jnp.float32)

    h1 = _tc_matmul(x, W1)
    as1, ad1, m1, m16 = _att_scalars(h1, att_src1, att_dst1, ae12[:, 0])
    num_p, den_p, deg_p, s1_p, s2_p = _SC_LAYER1(
        src3, dst3, ae1_att3, ones3, ae1_sum3, ae2_sum3,
        as1, ad1, m16, h1, z2, z1)
    num1 = (num_p[0] + num_p[1])[:N]
    den1 = (den_p[0] + den_p[1])[:N]
    deg = (deg_p[0] + deg_p[1])[:N]
    lae1 = ((s1_p[0] + s1_p[1])[:N]) / jnp.clip(deg, 1.0)
    lae2 = ((s2_p[0] + s2_p[1])[:N]) / jnp.clip(deg, 1.0)
    wl1 = jnp.exp(jax.nn.leaky_relu(as1 + ad1 + lae1, 0.2) - m1)
    o1 = (num1 + wl1[:, None] * h1) / (den1[:, None] + wl1[:, None] + 1e-16) + b1
    o1 = jax.nn.relu(o1)

    h2 = _tc_matmul(o1, W2)
    as2, ad2, m2, m16b = _att_scalars(h2, att_src2, att_dst2, ae12[:, 1])
    num_p2, den_p2 = _SC_LAYER2(src3, dst3, ae2_att3, as2, ad2, m16b, h2, z2, z1)
    num2 = (num_p2[0] + num_p2[1])[:N]
    den2 = (den_p2[0] + den_p2[1])[:N]
    wl2 = jnp.exp(jax.nn.leaky_relu(as2 + ad2 + lae2, 0.2) - m2)
    o2 = (num2 + wl2[:, None] * h2) / (den2[:, None] + wl2[:, None] + 1e-16) + b2

    out = o2 @ Wl + bl
    return jax.nn.relu(out)


# speedup vs baseline: 36.5242x; 1.7006x over previous
import dataclasses
import functools

import jax
import jax.numpy as jnp
from jax import lax
from jax.experimental import pallas as pl
from jax.experimental.pallas import tpu as pltpu
from jax.experimental.pallas import tpu_sc as plsc

N_NODES = 10000
N_EDGES = 320000
D_FEAT = 128
HID = 32

NC = 2
NS = 16
NW = NC * NS
CH = 128
NCH = 80
EC = NCH * CH
E_PAD = NW * EC
LAST_ROWS = (N_EDGES - (NW - 1) * EC) // CH
NP = 10240
NPS = NP // NS
NPN = N_NODES // NS


def _matmul_kernel(x_ref, w_ref, o_ref):
    o_ref[...] = jnp.dot(x_ref[...], w_ref[...], preferred_element_type=jnp.float32)


def _tc_matmul(x, w):
    n, k = x.shape
    m = w.shape[1]
    blk = 1000
    return pl.pallas_call(
        _matmul_kernel,
        grid=(n // blk,),
        in_specs=[
            pl.BlockSpec((blk, k), lambda i: (i, 0)),
            pl.BlockSpec((k, m), lambda i: (0, 0)),
        ],
        out_specs=pl.BlockSpec((blk, m), lambda i: (i, 0)),
        out_shape=jax.ShapeDtypeStruct((n, m), jnp.float32),
    )(x, w)


def _edge_proj_kernel(x_ref, p_ref, g_ref, o1_ref, o2_ref):
    hi = jax.lax.Precision.HIGHEST
    o1_ref[...] = jnp.dot(x_ref[...] * p_ref[0:1, :], g_ref[...],
                          preferred_element_type=jnp.float32, precision=hi)
    o2_ref[...] = jnp.dot(x_ref[...] * p_ref[1:2, :], g_ref[...],
                          preferred_element_type=jnp.float32, precision=hi)


def _edge_proj(ea128, p, g):
    n = ea128.shape[0]
    blk = 1000
    shp = jax.ShapeDtypeStruct((n, 32), jnp.float32)
    return pl.pallas_call(
        _edge_proj_kernel,
        grid=(n // blk,),
        in_specs=[
            pl.BlockSpec((blk, 128), lambda i: (i, 0)),
            pl.BlockSpec((2, 128), lambda i: (0, 0)),
            pl.BlockSpec((128, 32), lambda i: (0, 0)),
        ],
        out_specs=[pl.BlockSpec((blk, 32), lambda i: (i, 0))] * 2,
        out_shape=[shp, shp],
    )(ea128, p, g)


_MESH = plsc.VectorSubcoreMesh(core_axis_name="c", subcore_axis_name="s")

_SC_PARAMS = pltpu.CompilerParams()
if "needs_layout_passes" in pltpu.CompilerParams.__dataclass_fields__:
    _SC_PARAMS = dataclasses.replace(_SC_PARAMS, needs_layout_passes=False)
if "use_tc_tiling_on_sc" in pltpu.CompilerParams.__dataclass_fields__:
    _SC_PARAMS = dataclasses.replace(_SC_PARAMS, use_tc_tiling_on_sc=False)


def _make_sc_layer(with_deg):
    n_extra = 3 if with_deg else 0
    out_type = tuple([jax.ShapeDtypeStruct((NC, N_NODES, HID), jnp.float32)]
                     + [jax.ShapeDtypeStruct((NC, NP), jnp.float32)] * (1 + n_extra))
    scratch = [
        pltpu.VMEM((NCH, CH), jnp.int32),
        pltpu.VMEM((NCH, CH), jnp.int32),
        pltpu.VMEM((NCH, CH), jnp.float32),
        pltpu.VMEM((NCH, CH), jnp.float32),
        pltpu.VMEM((N_NODES,), jnp.float32),
        pltpu.VMEM((N_NODES,), jnp.float32),
        pltpu.VMEM((N_NODES,), jnp.float32),
        pltpu.VMEM((N_NODES,), jnp.float32),
        pltpu.VMEM((CH, HID), jnp.float32),
        pltpu.VMEM((CH, HID), jnp.float32),
        pltpu.VMEM_SHARED((N_NODES, HID), jnp.float32),
        pltpu.VMEM_SHARED((NP,), jnp.float32),
        pltpu.SemaphoreType.DMA,
        pltpu.SemaphoreType.DMA,
    ]
    if with_deg:
        scratch += [
            pltpu.VMEM((1, CH), jnp.float32),
            pltpu.VMEM((NCH, CH), jnp.float32),
            pltpu.VMEM((NCH, CH), jnp.float32),
            pltpu.VMEM_SHARED((NP,), jnp.float32),
            pltpu.VMEM_SHARED((NP,), jnp.float32),
            pltpu.VMEM_SHARED((NP,), jnp.float32),
        ]

    def body(*refs):
        if with_deg:
            (src_hbm, dst_hbm, e1_hbm, e2_hbm, ae1_hbm, ae2_hbm,
             eas_hbm, eas2_hbm, ead_hbm, ead2_hbm, h_hbm, z2_hbm, z1_hbm,
             num_out, den_out, deg_out, s1_out, s2_out,
             src_v, dst_v, w_v, e2_v, eas_v, eas2_v, ead_v, ead2_v,
             rows_a, rows_b, num_sh, den_sh, gsa, gsb,
             on_v, a1_v, a2_v, deg_sh, s1_sh, s2_sh) = refs
        else:
            (src_hbm, dst_hbm, e1_hbm, e2_hbm,
             eas_hbm, eas2_hbm, ead_hbm, ead2_hbm, h_hbm, z2_hbm, z1_hbm,
             num_out, den_out,
             src_v, dst_v, w_v, e2_v, eas_v, eas2_v, ead_v, ead2_v,
             rows_a, rows_b, num_sh, den_sh, gsa, gsb) = refs

        cid = lax.axis_index("c")
        sid = lax.axis_index("s")
        wid = cid * NS + sid
        nrows = jnp.where(wid == NW - 1, LAST_ROWS, NCH)

        pltpu.sync_copy(z2_hbm, num_sh.at[pl.ds(sid * NPN, NPN)])
        pltpu.sync_copy(z1_hbm, den_sh.at[pl.ds(sid * NPS, NPS)])
        if with_deg:
            pltpu.sync_copy(z1_hbm, deg_sh.at[pl.ds(sid * NPS, NPS)])
            pltpu.sync_copy(z1_hbm, s1_sh.at[pl.ds(sid * NPS, NPS)])
            pltpu.sync_copy(z1_hbm, s2_sh.at[pl.ds(sid * NPS, NPS)])

        pltpu.sync_copy(src_hbm.at[wid], src_v)
        pltpu.sync_copy(dst_hbm.at[wid], dst_v)
        pltpu.sync_copy(e1_hbm.at[wid], w_v)
        pltpu.sync_copy(e2_hbm.at[wid], e2_v)
        pltpu.sync_copy(eas_hbm, eas_v)
        pltpu.sync_copy(eas2_hbm, eas2_v)
        pltpu.sync_copy(ead_hbm, ead_v)
        pltpu.sync_copy(ead2_hbm, ead2_v)
        if with_deg:
            pltpu.sync_copy(ae1_hbm.at[wid], a1_v)
            pltpu.sync_copy(ae2_hbm.at[wid], a2_v)

            @pl.loop(0, CH // 16)
            def _ones(k):
                on_v[0, pl.ds(k * 16, 16)] = jnp.full((16,), 1.0, jnp.float32)
        plsc.subcore_barrier()

        @pl.loop(0, nrows)
        def _alpha(g):
            for k in range(8):
                o = k * 16
                s16 = src_v[g, pl.ds(o, 16)]
                d16 = dst_v[g, pl.ds(o, 16)]
                wpos = (plsc.load_gather(eas_v, [s16])
                        * plsc.load_gather(ead_v, [d16]) * w_v[g, pl.ds(o, 16)])
                wneg = (plsc.load_gather(eas2_v, [s16])
                        * plsc.load_gather(ead2_v, [d16]) * e2_v[g, pl.ds(o, 16)])
                w_v[g, pl.ds(o, 16)] = jnp.maximum(wpos, wneg)

        @pl.loop(0, nrows)
        def _scal_scat(c):
            pltpu.sync_copy(w_v.at[c], den_sh.at[dst_v.at[c]], add=True)
            if with_deg:
                pltpu.sync_copy(on_v.at[0], deg_sh.at[dst_v.at[c]], add=True)
                pltpu.sync_copy(a1_v.at[c], s1_sh.at[dst_v.at[c]], add=True)
                pltpu.sync_copy(a2_v.at[c], s2_sh.at[dst_v.at[c]], add=True)

        def scale(rows_v, c):
            @pl.loop(0, CH // 8)
            def _scale(eo):
                for j in range(8):
                    e = eo * 8 + j
                    wv16 = plsc.load_gather(
                        w_v, [jnp.full((16,), c, jnp.int32),
                              jnp.full((16,), e, jnp.int32)])
                    rows_v[e, pl.ds(0, 16)] = rows_v[e, pl.ds(0, 16)] * wv16
                    rows_v[e, pl.ds(16, 16)] = rows_v[e, pl.ds(16, 16)] * wv16

        pltpu.async_copy(h_hbm.at[src_v.at[0]], rows_a, gsa)
        pltpu.async_copy(h_hbm.at[src_v.at[1]], rows_b, gsb)

        @pl.loop(0, nrows // 2)
        def _agg(p):
            c0 = p * 2
            c1 = c0 + 1

            pltpu.make_async_copy(h_hbm.at[src_v.at[c0]], rows_a, gsa).wait()
            scale(rows_a, c0)
            pltpu.sync_copy(rows_a, num_sh.at[dst_v.at[c0]], add=True)

            @pl.when(c0 + 2 < nrows)
            def _next_a():
                pltpu.async_copy(h_hbm.at[src_v.at[c0 + 2]], rows_a, gsa)

            pltpu.make_async_copy(h_hbm.at[src_v.at[c1]], rows_b, gsb).wait()
            scale(rows_b, c1)
            pltpu.sync_copy(rows_b, num_sh.at[dst_v.at[c1]], add=True)

            @pl.when(c1 + 2 < nrows)
            def _next_b():
                pltpu.async_copy(h_hbm.at[src_v.at[c1 + 2]], rows_b, gsb)

        plsc.subcore_barrier()

        sl = pl.ds(sid * NPS, NPS)
        sln = pl.ds(sid * NPN, NPN)
        pltpu.sync_copy(num_sh.at[sln], num_out.at[cid, sln])
        pltpu.sync_copy(den_sh.at[sl], den_out.at[cid, sl])
        if with_deg:
            pltpu.sync_copy(deg_sh.at[sl], deg_out.at[cid, sl])
            pltpu.sync_copy(s1_sh.at[sl], s1_out.at[cid, sl])
            pltpu.sync_copy(s2_sh.at[sl], s2_out.at[cid, sl])

    return functools.partial(
        pl.kernel, mesh=_MESH, out_type=out_type,
        compiler_params=_SC_PARAMS, scratch_types=scratch)(body)


_SC_LAYER = _make_sc_layer(False)


@functools.partial(
    pl.kernel,
    mesh=_MESH,
    out_type=(jax.ShapeDtypeStruct((NC, NP), jnp.float32),
              jax.ShapeDtypeStruct((NC, NP), jnp.float32),
              jax.ShapeDtypeStruct((NC, NP), jnp.float32)),
    compiler_params=_SC_PARAMS,
    scratch_types=[
        pltpu.VMEM((NCH, CH), jnp.int32),
        pltpu.VMEM((1, CH), jnp.float32),
        pltpu.VMEM((NCH, CH), jnp.float32),
        pltpu.VMEM((NCH, CH), jnp.float32),
        pltpu.VMEM_SHARED((NP,), jnp.float32),
        pltpu.VMEM_SHARED((NP,), jnp.float32),
        pltpu.VMEM_SHARED((NP,), jnp.float32),
    ],
)
def _sc_deg_kernel(dst_hbm, ae1_hbm, ae2_hbm, z1_hbm,
                   deg_out, s1_out, s2_out,
                   dst_v, on_v, a1_v, a2_v, deg_sh, s1_sh, s2_sh):
    cid = lax.axis_index("c")
    sid = lax.axis_index("s")
    wid = cid * NS + sid
    nrows = jnp.where(wid == NW - 1, LAST_ROWS, NCH)

    pltpu.sync_copy(z1_hbm, deg_sh.at[pl.ds(sid * NPS, NPS)])
    pltpu.sync_copy(z1_hbm, s1_sh.at[pl.ds(sid * NPS, NPS)])
    pltpu.sync_copy(z1_hbm, s2_sh.at[pl.ds(sid * NPS, NPS)])

    pltpu.sync_copy(dst_hbm.at[wid], dst_v)
    pltpu.sync_copy(ae1_hbm.at[wid], a1_v)
    pltpu.sync_copy(ae2_hbm.at[wid], a2_v)

    @pl.loop(0, CH // 16)
    def _ones(k):
        on_v[0, pl.ds(k * 16, 16)] = jnp.full((16,), 1.0, jnp.float32)

    plsc.subcore_barrier()

    @pl.loop(0, nrows)
    def _scat(c):
        pltpu.sync_copy(on_v.at[0], deg_sh.at[dst_v.at[c]], add=True)
        pltpu.sync_copy(a1_v.at[c], s1_sh.at[dst_v.at[c]], add=True)
        pltpu.sync_copy(a2_v.at[c], s2_sh.at[dst_v.at[c]], add=True)

    plsc.subcore_barrier()

    sl = pl.ds(sid * NPS, NPS)
    pltpu.sync_copy(deg_sh.at[sl], deg_out.at[cid, sl])
    pltpu.sync_copy(s1_sh.at[sl], s1_out.at[cid, sl])
    pltpu.sync_copy(s2_sh.at[sl], s2_out.at[cid, sl])


def _pad_edges(a, fill):
    pad = E_PAD - N_EDGES
    return jnp.concatenate([a, jnp.full((pad,), fill, a.dtype)]).reshape(NW, NCH, CH)


def _att_exp(h, att_src, att_dst, ae):
    a_s = (h * att_src).sum(-1)
    a_d = (h * att_dst).sum(-1)
    m = jax.nn.leaky_relu(jnp.max(a_s) + jnp.max(a_d)
                          + jnp.maximum(jnp.max(ae), 0.0), 0.2)
    exps = (jnp.exp(a_s), jnp.exp(0.2 * a_s), jnp.exp(a_d), jnp.exp(0.2 * a_d))
    e1 = _pad_edges(jnp.exp(ae - m), 0.0)
    e2 = _pad_edges(jnp.exp(0.2 * ae - m), 0.0)
    return a_s, a_d, m, exps, e1, e2


def kernel(x, edge_index, edge_attr, W1, att_src1, att_dst1, We1, att_e1, b1,
           W2, att_src2, att_dst2, We2, att_e2, b2, Wl, bl):
    N = N_NODES
    src, dst = edge_index[0], edge_index[1]
    src3 = _pad_edges(src, 0)
    dst3 = _pad_edges(dst, 0)

    wv = jnp.stack([We1 @ att_e1, We2 @ att_e2], axis=1)
    f_idx = jnp.tile(jnp.arange(4), 32)
    p = wv[f_idx, :].T
    g = (jnp.arange(128)[:, None] // 4
         == jnp.arange(32)[None, :]).astype(jnp.float32)
    ea128 = edge_attr.reshape(N_EDGES // 32, 128)
    ae1_2d, ae2_2d = _edge_proj(ea128, p, g)
    ae1 = ae1_2d.reshape(-1)
    ae2 = ae2_2d.reshape(-1)
    ae1_3 = _pad_edges(ae1, 0.0)
    ae2_3 = _pad_edges(ae2, 0.0)

    z2 = jnp.zeros((NPN, HID), jnp.float32)
    z1 = jnp.zeros((NPS,), jnp.float32)

    h1 = _tc_matmul(x, W1)
    as1, ad1, m1, exps1, e1a, e2a = _att_exp(h1, att_src1, att_dst1, ae1)
    deg_p, s1_p, s2_p = _sc_deg_kernel(dst3, ae1_3, ae2_3, z1)
    num_p, den_p = _SC_LAYER(src3, dst3, e1a, e2a, *exps1, h1, z2, z1)
    num1 = num_p[0] + num_p[1]
    den1 = (den_p[0] + den_p[1])[:N]
    deg = (deg_p[0] + deg_p[1])[:N]
    lae1 = ((s1_p[0] + s1_p[1])[:N]) / jnp.clip(deg, 1.0)
    lae2 = ((s2_p[0] + s2_p[1])[:N]) / jnp.clip(deg, 1.0)
    wl1 = jnp.exp(jax.nn.leaky_relu(as1 + ad1 + lae1, 0.2) - m1)
    o1 = (num1 + wl1[:, None] * h1) / (den1[:, None] + wl1[:, None] + 1e-16) + b1
    o1 = jax.nn.relu(o1)

    h2 = _tc_matmul(o1, W2)
    as2, ad2, m2, exps2, e1b, e2b = _att_exp(h2, att_src2, att_dst2, ae2)
    num_p2, den_p2 = _SC_LAYER(src3, dst3, e1b, e2b, *exps2, h2, z2, z1)
    num2 = num_p2[0] + num_p2[1]
    den2 = (den_p2[0] + den_p2[1])[:N]
    wl2 = jnp.exp(jax.nn.leaky_relu(as2 + ad2 + lae2, 0.2) - m2)
    o2 = (num2 + wl2[:, None] * h2) / (den2[:, None] + wl2[:, None] + 1e-16) + b2

    out = o2 @ Wl + bl
    return jax.nn.relu(out)
